# baseline (speedup 1.0000x reference)
import functools

import jax
import jax.numpy as jnp
from jax import lax
from jax.experimental import pallas as pl
from jax.experimental.pallas import tpu as pltpu
from jax.experimental.pallas import tpu_sc as plsc

NUM_CORES = 2
NUM_SUBCORES = 16
NUM_WORKERS = NUM_CORES * NUM_SUBCORES
MAX_IDX = 128
CHUNK_ROWS = 1
NBUF = 8
GATHER_LAT = 5


@functools.lru_cache(maxsize=None)
def _build_gather(batch: int, hist: int, vocab: int, d_model: int):
    assert batch % NUM_WORKERS == 0
    rows_per_w = batch // NUM_WORKERS
    assert rows_per_w % (CHUNK_ROWS * NBUF) == 0
    n_chunks = rows_per_w // CHUNK_ROWS
    n_groups = n_chunks // NBUF
    splits = []
    off = 0
    while off < hist:
        w = min(MAX_IDX, hist - off)
        splits.append((off, w))
        off += w

    mesh = plsc.VectorSubcoreMesh(core_axis_name="c", subcore_axis_name="s")

    @functools.partial(
        pl.kernel,
        mesh=mesh,
        out_type=jax.ShapeDtypeStruct((batch, hist, d_model), jnp.float32),
        scratch_types=[
            pltpu.VMEM((NBUF * CHUNK_ROWS, hist), jnp.int32),
            pltpu.VMEM((NBUF * CHUNK_ROWS, hist, d_model), jnp.float32),
        ]
        + [pltpu.SemaphoreType.DMA] * (2 * NBUF),
        compiler_params=pltpu.CompilerParams(use_tc_tiling_on_sc=False),
    )
    def gather_kernel(table_hbm, idx_hbm, out_hbm, idx_v, rows_v, *sems):
        sem_g = sems[:NBUF]
        sem_s = sems[NBUF:]
        wid = lax.axis_index("s") * NUM_CORES + lax.axis_index("c")
        row0 = wid * rows_per_w

        def load_idx(i, b):
            pltpu.sync_copy(
                idx_hbm.at[pl.ds(row0 + i * CHUNK_ROWS, CHUNK_ROWS)],
                idx_v.at[pl.ds(b * CHUNK_ROWS, CHUNK_ROWS)],
            )

        def gather_copies(b):
            for j in range(CHUNK_ROWS):
                r = b * CHUNK_ROWS + j
                for off, w in splits:
                    yield pltpu.make_async_copy(
                        table_hbm.at[idx_v.at[r, pl.ds(off, w)]],
                        rows_v.at[r, pl.ds(off, w)],
                        sem_g[b],
                    )

        def start_gather(b):
            for cp in gather_copies(b):
                cp.start()

        def wait_gather(b):
            for cp in gather_copies(b):
                cp.wait()

        def store_copy(i, b):
            return pltpu.make_async_copy(
                rows_v.at[pl.ds(b * CHUNK_ROWS, CHUNK_ROWS)],
                out_hbm.at[pl.ds(row0 + i * CHUNK_ROWS, CHUNK_ROWS)],
                sem_s[b],
            )

        for i in range(NBUF):
            load_idx(i, i)
            start_gather(i)
            if i >= GATHER_LAT:
                c = i - GATHER_LAT
                wait_gather(c % NBUF)
                store_copy(c, c % NBUF).start()

        def group(g, carry):
            for b in range(NBUF):
                i = g * NBUF + b
                store_copy(i - NBUF, b).wait()
                load_idx(i, b)
                start_gather(b)
                bd = (b - GATHER_LAT) % NBUF
                wait_gather(bd)
                store_copy(i - GATHER_LAT, bd).start()
            return carry

        lax.fori_loop(1, n_groups, group, 0)

        for c in range(n_chunks - GATHER_LAT, n_chunks):
            wait_gather(c % NBUF)
            store_copy(c, c % NBUF).start()
        for c in range(n_chunks - NBUF, n_chunks):
            store_copy(c, c % NBUF).wait()

    return gather_kernel


def kernel(x, table):
    batch, hist = x.shape
    vocab, d_model = table.shape
    out = _build_gather(batch, hist, vocab, d_model)(
        table, x.astype(jnp.int32)
    )
    return out
